# baseline (speedup 1.0000x reference)
import functools

import jax
import jax.numpy as jnp
from jax import lax
from jax.experimental import pallas as pl
from jax.experimental.pallas import tpu as pltpu
from jax.experimental.pallas import tpu_sc as plsc

_C = 80
_B = 16
_N = 8400
_T = 32
_LAMBDA_COORD = 5.0


def _softplus(x):
    return jnp.maximum(x, 0.0) + jnp.log1p(jnp.exp(-jnp.abs(x)))


_ATAN_COEFFS = (9.999998711639e-01, -3.333252400263e-01, 1.998488468557e-01,
                -1.415480604185e-01, 1.047753919858e-01, -7.194384542314e-02,
                3.934541314624e-02, -1.415234803516e-02, 2.398139012257e-03)


def _atan(x):
    a = jnp.abs(x)
    inv = a > 1.0
    z = jnp.where(inv, 1.0 / a, a)
    t = z * z
    p = jnp.full_like(z, _ATAN_COEFFS[-1])
    for coef in _ATAN_COEFFS[-2::-1]:
        p = p * t + coef
    p = p * z
    r = jnp.where(inv, (jnp.pi / 2.0) - p, p)
    return jnp.sign(x) * r


def _match_body(px1_ref, py1_ref, px2_ref, py2_ref, pc_ref, gt_ref,
                bl_ref, ol_ref, idx_ref):
    b = pl.program_id(0)
    px1 = px1_ref[0]
    py1 = py1_ref[0]
    px2 = px2_ref[0]
    py2 = py2_ref[0]
    pc = pc_ref[0]
    gt = gt_ref[0]
    gx1 = gt[:, 0:1]
    gy1 = gt[:, 1:2]
    gx2 = gt[:, 2:3]
    gy2 = gt[:, 3:4]

    parea = (px2 - px1) * (py2 - py1)
    garea = (gx2 - gx1) * (gy2 - gy1)
    ix1 = jnp.maximum(px1, gx1)
    iy1 = jnp.maximum(py1, gy1)
    ix2 = jnp.minimum(px2, gx2)
    iy2 = jnp.minimum(py2, gy2)
    inter = jnp.maximum(ix2 - ix1, 0.0) * jnp.maximum(iy2 - iy1, 0.0)
    union = parea + garea - inter
    iou = inter / (union + 1e-6)

    m = jnp.max(iou, axis=1, keepdims=True)
    lane = lax.broadcasted_iota(jnp.int32, iou.shape, 1)
    best = jnp.min(jnp.where(iou == m, lane, _N - 1), axis=1,
                   keepdims=True)

    onehot = lane == best
    colmask = jnp.max(jnp.where(onehot, 1.0, 0.0), axis=0, keepdims=True)
    obj = (jnp.sum(_softplus(pc)) - jnp.sum(colmask * pc)) / float(_N)
    ol_ref[0] = jnp.broadcast_to(obj, (1, 1))

    mx1 = jnp.sum(jnp.where(onehot, px1, 0.0), axis=1, keepdims=True)
    my1 = jnp.sum(jnp.where(onehot, py1, 0.0), axis=1, keepdims=True)
    mx2 = jnp.sum(jnp.where(onehot, px2, 0.0), axis=1, keepdims=True)
    my2 = jnp.sum(jnp.where(onehot, py2, 0.0), axis=1, keepdims=True)

    area1 = (mx2 - mx1) * (my2 - my1)
    left = jnp.maximum(mx1, gx1)
    top = jnp.maximum(my1, gy1)
    right = jnp.minimum(mx2, gx2)
    bottom = jnp.minimum(my2, gy2)
    wh = jnp.maximum(right - left, 0.0) * jnp.maximum(bottom - top, 0.0)
    uni = area1 + garea - wh
    iou_d = wh / (uni + 1e-6)
    cx1 = (mx1 + mx2) * 0.5
    cy1 = (my1 + my2) * 0.5
    cx2 = (gx1 + gx2) * 0.5
    cy2 = (gy1 + gy2) * 0.5
    ex1 = jnp.minimum(mx1, gx1)
    ey1 = jnp.minimum(my1, gy1)
    ex2 = jnp.maximum(mx2, gx2)
    ey2 = jnp.maximum(my2, gy2)
    c_diag = (ex2 - ex1) ** 2 + (ey2 - ey1) ** 2
    center_dist = (cx1 - cx2) ** 2 + (cy1 - cy2) ** 2
    w1 = mx2 - mx1
    h1 = my2 - my1
    w2 = gx2 - gx1
    h2 = gy2 - gy1
    v = 4.0 / (jnp.pi ** 2) * (_atan(w2 / h2) - _atan(w1 / h1)) ** 2
    alpha = v / (1.0 - iou_d + v + 1e-6)
    ciou = iou_d - center_dist / c_diag - alpha * v
    bl_ref[0] = jnp.broadcast_to(jnp.sum(1.0 - ciou) / float(_T), (1, 1))

    ident = (lax.broadcasted_iota(jnp.int32, (_T, _T), 0)
             == lax.broadcasted_iota(jnp.int32, (_T, _T), 1))
    best_t = jnp.sum(jnp.where(ident, jnp.broadcast_to(best, (_T, _T)), 0),
                     axis=0, keepdims=True)
    idx_ref[0] = best_t + b * _N


def _match(px1, py1, px2, py2, pc, gt):
    spec_n = pl.BlockSpec((1, 1, _N), lambda b: (b, 0, 0))
    spec_s = pl.BlockSpec((1, 1, 1), lambda b: (b, 0, 0))
    return pl.pallas_call(
        _match_body,
        grid=(_B,),
        in_specs=[spec_n, spec_n, spec_n, spec_n, spec_n,
                  pl.BlockSpec((1, _T, 4), lambda b: (b, 0, 0))],
        out_specs=[spec_s, spec_s,
                   pl.BlockSpec((1, 1, _T), lambda b: (b, 0, 0))],
        out_shape=[jax.ShapeDtypeStruct((_B, 1, 1), jnp.float32),
                   jax.ShapeDtypeStruct((_B, 1, 1), jnp.float32),
                   jax.ShapeDtypeStruct((_B, 1, _T), jnp.int32)],
        compiler_params=pltpu.CompilerParams(
            dimension_semantics=("arbitrary",)),
    )(px1, py1, px2, py2, pc, gt)


def _gather_rows_sc(table, idx):
    n_idx = idx.shape[0]
    nw = 32
    b_per_w = n_idx // nw
    mesh = plsc.VectorSubcoreMesh(core_axis_name="c", subcore_axis_name="s")

    @functools.partial(
        pl.kernel, mesh=mesh,
        out_type=jax.ShapeDtypeStruct((n_idx, _C), jnp.float32),
        scratch_types=[
            pltpu.VMEM((b_per_w,), jnp.int32),
            pltpu.VMEM((b_per_w, _C), jnp.float32),
            pltpu.SemaphoreType.DMA,
        ],
    )
    def k(table_hbm, idx_hbm, out_hbm, idx_v, rows_v, sem):
        wid = lax.axis_index("s") * 2 + lax.axis_index("c")
        base = wid * b_per_w
        pltpu.sync_copy(idx_hbm.at[pl.ds(base, b_per_w)], idx_v)
        iv = idx_v[...]
        cps = []
        for j in range(b_per_w):
            r = iv[j]
            cps.append(pltpu.async_copy(
                table_hbm.at[pl.ds(r, 1)], rows_v.at[pl.ds(j, 1)], sem))
        for c in cps:
            c.wait()
        pltpu.sync_copy(rows_v, out_hbm.at[pl.ds(base, b_per_w)])

    return k(table, idx)


def _finish_body(rows_ref, lab_ref, bl_ref, ol_ref, out_ref):
    x = rows_ref[...]
    lab = lab_ref[...]
    cls_iota = lax.broadcasted_iota(jnp.int32, x.shape, 1)
    picked = jnp.sum(jnp.where(cls_iota == lab, x, 0.0))
    sp = jnp.sum(_softplus(x))
    cls_total = (sp - picked) / float(_T * _C * _B)
    box_total = jnp.sum(bl_ref[...]) / float(_B)
    obj_total = jnp.sum(ol_ref[...]) / float(_B)
    out_ref[...] = jnp.broadcast_to(
        _LAMBDA_COORD * box_total + obj_total + cls_total, (1, 1))


def _finish(rows, labels2, bl, ol):
    return pl.pallas_call(
        _finish_body,
        out_shape=jax.ShapeDtypeStruct((1, 1), jnp.float32),
    )(rows, labels2, bl, ol)


def kernel(pred_boxes, pred_conf, pred_cls, boxes, labels, anchors):
    del anchors
    px1 = pred_boxes[..., 0].reshape(_B, 1, _N)
    py1 = pred_boxes[..., 1].reshape(_B, 1, _N)
    px2 = pred_boxes[..., 2].reshape(_B, 1, _N)
    py2 = pred_boxes[..., 3].reshape(_B, 1, _N)
    pc = pred_conf[..., 0].reshape(_B, 1, _N)
    bl, ol, idx = _match(px1, py1, px2, py2, pc, boxes)
    rows = _gather_rows_sc(pred_cls.reshape(_B * _N, _C),
                           idx.reshape(_B * _T))
    out = _finish(rows, labels.reshape(_B * _T, 1).astype(jnp.int32),
                  bl.reshape(_B, 1), ol.reshape(_B, 1))
    return out.reshape(())


# speedup vs baseline: 2.7439x; 2.7439x over previous
import functools

import jax
import jax.numpy as jnp
from jax import lax
from jax.experimental import pallas as pl
from jax.experimental.pallas import tpu as pltpu
from jax.experimental.pallas import tpu_sc as plsc

_C = 80
_B = 16
_N = 8400
_T = 32
_NW = 32
_BPW = (_B * _T) // _NW
_LAMBDA_COORD = 5.0


def _softplus(x):
    return jnp.maximum(x, 0.0) + jnp.log1p(jnp.exp(-jnp.abs(x)))


_ATAN_COEFFS = (9.999998711639e-01, -3.333252400263e-01, 1.998488468557e-01,
                -1.415480604185e-01, 1.047753919858e-01, -7.194384542314e-02,
                3.934541314624e-02, -1.415234803516e-02, 2.398139012257e-03)


def _atan(x):
    a = jnp.abs(x)
    inv = a > 1.0
    z = jnp.where(inv, 1.0 / a, a)
    t = z * z
    p = jnp.full_like(z, _ATAN_COEFFS[-1])
    for coef in _ATAN_COEFFS[-2::-1]:
        p = p * t + coef
    p = p * z
    r = jnp.where(inv, (jnp.pi / 2.0) - p, p)
    return jnp.sign(x) * r


def _match_body(pb_ref, pc_ref, gt_ref, bl_ref, ol_ref, idx_ref):
    b = pl.program_id(0)
    px1 = pb_ref[pl.ds(4 * b + 0, 1), :]
    py1 = pb_ref[pl.ds(4 * b + 1, 1), :]
    px2 = pb_ref[pl.ds(4 * b + 2, 1), :]
    py2 = pb_ref[pl.ds(4 * b + 3, 1), :]
    pc = pc_ref[pl.ds(b, 1), :]
    gt = gt_ref[0]
    gx1 = gt[:, 0:1]
    gy1 = gt[:, 1:2]
    gx2 = gt[:, 2:3]
    gy2 = gt[:, 3:4]

    parea = (px2 - px1) * (py2 - py1)
    garea = (gx2 - gx1) * (gy2 - gy1)
    ix1 = jnp.maximum(px1, gx1)
    iy1 = jnp.maximum(py1, gy1)
    ix2 = jnp.minimum(px2, gx2)
    iy2 = jnp.minimum(py2, gy2)
    inter = jnp.maximum(ix2 - ix1, 0.0) * jnp.maximum(iy2 - iy1, 0.0)
    union = parea + garea - inter
    iou = inter / (union + 1e-6)

    m = jnp.max(iou, axis=1, keepdims=True)
    lane = lax.broadcasted_iota(jnp.int32, iou.shape, 1)
    best = jnp.min(jnp.where(iou == m, lane, _N - 1), axis=1,
                   keepdims=True)

    onehot = lane == best
    colmask = jnp.max(jnp.where(onehot, 1.0, 0.0), axis=0, keepdims=True)
    obj = (jnp.sum(_softplus(pc)) - jnp.sum(colmask * pc)) / float(_N)
    ol_ref[0] = jnp.broadcast_to(obj, (1, 1))

    mx1 = jnp.sum(jnp.where(onehot, px1, 0.0), axis=1, keepdims=True)
    my1 = jnp.sum(jnp.where(onehot, py1, 0.0), axis=1, keepdims=True)
    mx2 = jnp.sum(jnp.where(onehot, px2, 0.0), axis=1, keepdims=True)
    my2 = jnp.sum(jnp.where(onehot, py2, 0.0), axis=1, keepdims=True)

    area1 = (mx2 - mx1) * (my2 - my1)
    left = jnp.maximum(mx1, gx1)
    top = jnp.maximum(my1, gy1)
    right = jnp.minimum(mx2, gx2)
    bottom = jnp.minimum(my2, gy2)
    wh = jnp.maximum(right - left, 0.0) * jnp.maximum(bottom - top, 0.0)
    uni = area1 + garea - wh
    iou_d = wh / (uni + 1e-6)
    cx1 = (mx1 + mx2) * 0.5
    cy1 = (my1 + my2) * 0.5
    cx2 = (gx1 + gx2) * 0.5
    cy2 = (gy1 + gy2) * 0.5
    ex1 = jnp.minimum(mx1, gx1)
    ey1 = jnp.minimum(my1, gy1)
    ex2 = jnp.maximum(mx2, gx2)
    ey2 = jnp.maximum(my2, gy2)
    c_diag = (ex2 - ex1) ** 2 + (ey2 - ey1) ** 2
    center_dist = (cx1 - cx2) ** 2 + (cy1 - cy2) ** 2
    w1 = mx2 - mx1
    h1 = my2 - my1
    w2 = gx2 - gx1
    h2 = gy2 - gy1
    v = 4.0 / (jnp.pi ** 2) * (_atan(w2 / h2) - _atan(w1 / h1)) ** 2
    alpha = v / (1.0 - iou_d + v + 1e-6)
    ciou = iou_d - center_dist / c_diag - alpha * v
    bl_ref[0] = jnp.broadcast_to(jnp.sum(1.0 - ciou) / float(_T), (1, 1))

    ident = (lax.broadcasted_iota(jnp.int32, (_T, _T), 0)
             == lax.broadcasted_iota(jnp.int32, (_T, _T), 1))
    best_t = jnp.sum(jnp.where(ident, jnp.broadcast_to(best, (_T, _T)), 0),
                     axis=0, keepdims=True)
    idx_ref[0] = best_t


def _match(pb, pc, gt):
    return pl.pallas_call(
        _match_body,
        grid=(_B,),
        in_specs=[pl.BlockSpec((4 * _B, _N), lambda b: (0, 0)),
                  pl.BlockSpec((_B, _N), lambda b: (0, 0)),
                  pl.BlockSpec((1, _T, 4), lambda b: (b, 0, 0))],
        out_specs=[pl.BlockSpec((1, 1, 1), lambda b: (b, 0, 0)),
                   pl.BlockSpec((1, 1, 1), lambda b: (b, 0, 0)),
                   pl.BlockSpec((1, 1, _T), lambda b: (b, 0, 0))],
        out_shape=[jax.ShapeDtypeStruct((_B, 1, 1), jnp.float32),
                   jax.ShapeDtypeStruct((_B, 1, 1), jnp.float32),
                   jax.ShapeDtypeStruct((_B, 1, _T), jnp.int32)],
        compiler_params=pltpu.CompilerParams(
            dimension_semantics=("arbitrary",)),
    )(pb, pc, gt)


def _gather_cols_sc(table, idx):
    mesh = plsc.VectorSubcoreMesh(core_axis_name="c", subcore_axis_name="s")

    nbuf = 4

    @functools.partial(
        pl.kernel, mesh=mesh,
        out_type=jax.ShapeDtypeStruct((_NW, _C, _BPW), jnp.float32),
        scratch_types=[
            pltpu.VMEM((_BPW,), jnp.int32),
            pltpu.VMEM((nbuf, _C, 128), jnp.float32),
            pltpu.VMEM((_C, _BPW), jnp.float32),
            pltpu.SemaphoreType.DMA,
        ],
        compiler_params=pltpu.CompilerParams(needs_layout_passes=False),
    )
    def k(table_hbm, idx_hbm, out_hbm, idx_v, slab_v, col_v, sem):
        wid = lax.axis_index("s") * 2 + lax.axis_index("c")
        base = wid * _BPW
        pltpu.sync_copy(idx_hbm.at[pl.ds(base, _BPW)], idx_v)
        iv = idx_v[...]
        riota = lax.iota(jnp.int32, 16)

        def fire(j):
            row0 = pl.multiple_of(((base + j) // _T) * _C, _C)
            r = iv[j]
            lt = pl.multiple_of(r - r % 128, 128)
            return pltpu.async_copy(
                table_hbm.at[pl.ds(row0, _C), pl.ds(lt, 128)],
                slab_v.at[j % nbuf], sem)

        def select(j):
            rmv = jnp.full((16,), iv[j] % 128, dtype=jnp.int32)
            for c in range(_C // 16):
                vals = plsc.load_gather(
                    slab_v.at[j % nbuf], [c * 16 + riota, rmv])
                plsc.store_scatter(
                    col_v, [c * 16 + riota,
                            jnp.full((16,), j, dtype=jnp.int32)], vals)

        cps = [fire(j) for j in range(nbuf)]
        for j in range(_BPW):
            cps[j].wait()
            select(j)
            if j + nbuf < _BPW:
                cps.append(fire(j + nbuf))
        pltpu.sync_copy(col_v, out_hbm.at[wid])

    return k(table, idx)


def _finish_body(x_ref, lab_ref, bl_ref, ol_ref, out_ref):
    x = x_ref[...]
    lab = lab_ref[...]
    cls_iota = lax.broadcasted_iota(jnp.int32, x.shape, 1)
    picked = jnp.sum(jnp.where(cls_iota == lab, x, 0.0))
    sp = jnp.sum(_softplus(x))
    cls_total = (sp - picked) / float(_T * _C * _B)
    box_total = jnp.sum(bl_ref[...]) / float(_B)
    obj_total = jnp.sum(ol_ref[...]) / float(_B)
    out_ref[...] = jnp.broadcast_to(
        _LAMBDA_COORD * box_total + obj_total + cls_total, (1, 1))


def _finish(x, labg, bl, ol):
    return pl.pallas_call(
        _finish_body,
        out_shape=jax.ShapeDtypeStruct((1, 1), jnp.float32),
    )(x, labg, bl, ol)


def kernel(pred_boxes, pred_conf, pred_cls, boxes, labels, anchors):
    del anchors
    pb = jnp.swapaxes(pred_boxes, 1, 2).reshape(4 * _B, _N)
    pc = jnp.swapaxes(pred_conf, 1, 2).reshape(_B, _N)
    bl, ol, idx = _match(pb, pc, boxes)
    table = jnp.swapaxes(pred_cls, 1, 2).reshape(_B * _C, _N)
    cols = _gather_cols_sc(table, idx.reshape(_B * _T))
    labg = labels.reshape(_NW, 1, _BPW).astype(jnp.int32)
    out = _finish(cols, labg, bl.reshape(_B, 1), ol.reshape(_B, 1))
    return out.reshape(())


# speedup vs baseline: 2.8188x; 1.0273x over previous
import functools

import jax
import jax.numpy as jnp
from jax import lax
from jax.experimental import pallas as pl
from jax.experimental.pallas import tpu as pltpu
from jax.experimental.pallas import tpu_sc as plsc

_C = 80
_B = 16
_N = 8400
_T = 32
_NW = 32
_BPW = (_B * _T) // _NW
_LAMBDA_COORD = 5.0


def _softplus(x):
    return jnp.maximum(x, 0.0) + jnp.log1p(jnp.exp(-jnp.abs(x)))


_ATAN_COEFFS = (9.999998711639e-01, -3.333252400263e-01, 1.998488468557e-01,
                -1.415480604185e-01, 1.047753919858e-01, -7.194384542314e-02,
                3.934541314624e-02, -1.415234803516e-02, 2.398139012257e-03)


def _atan(x):
    a = jnp.abs(x)
    inv = a > 1.0
    z = jnp.where(inv, 1.0 / a, a)
    t = z * z
    p = jnp.full_like(z, _ATAN_COEFFS[-1])
    for coef in _ATAN_COEFFS[-2::-1]:
        p = p * t + coef
    p = p * z
    r = jnp.where(inv, (jnp.pi / 2.0) - p, p)
    return jnp.sign(x) * r


def _argmax_body(pb_ref, gt_ref, idx_ref, idx2_ref):
    b = pl.program_id(0)
    px1 = pb_ref[pl.ds(4 * b + 0, 1), :]
    py1 = pb_ref[pl.ds(4 * b + 1, 1), :]
    px2 = pb_ref[pl.ds(4 * b + 2, 1), :]
    py2 = pb_ref[pl.ds(4 * b + 3, 1), :]
    gt = gt_ref[0]
    gx1 = gt[:, 0:1]
    gy1 = gt[:, 1:2]
    gx2 = gt[:, 2:3]
    gy2 = gt[:, 3:4]

    parea = (px2 - px1) * (py2 - py1)
    garea = (gx2 - gx1) * (gy2 - gy1)
    ix1 = jnp.maximum(px1, gx1)
    iy1 = jnp.maximum(py1, gy1)
    ix2 = jnp.minimum(px2, gx2)
    iy2 = jnp.minimum(py2, gy2)
    inter = jnp.maximum(ix2 - ix1, 0.0) * jnp.maximum(iy2 - iy1, 0.0)
    union = parea + garea - inter
    iou = inter / (union + 1e-6)

    m = jnp.max(iou, axis=1, keepdims=True)
    lane = lax.broadcasted_iota(jnp.int32, iou.shape, 1)
    best = jnp.min(jnp.where(iou == m, lane, _N - 1), axis=1,
                   keepdims=True)
    idx2_ref[0] = best

    ident = (lax.broadcasted_iota(jnp.int32, (_T, _T), 0)
             == lax.broadcasted_iota(jnp.int32, (_T, _T), 1))
    best_t = jnp.sum(jnp.where(ident, jnp.broadcast_to(best, (_T, _T)), 0),
                     axis=0, keepdims=True)
    idx_ref[0] = best_t


def _argmax(pb, gt):
    return pl.pallas_call(
        _argmax_body,
        grid=(_B,),
        in_specs=[pl.BlockSpec((4 * _B, _N), lambda b: (0, 0)),
                  pl.BlockSpec((1, _T, 4), lambda b: (b, 0, 0))],
        out_specs=[pl.BlockSpec((1, 1, _T), lambda b: (b, 0, 0)),
                   pl.BlockSpec((1, _T, 1), lambda b: (b, 0, 0))],
        out_shape=[jax.ShapeDtypeStruct((_B, 1, _T), jnp.int32),
                   jax.ShapeDtypeStruct((_B, _T, 1), jnp.int32)],
        compiler_params=pltpu.CompilerParams(
            dimension_semantics=("arbitrary",)),
    )(pb, gt)


def _losses_body(pb_ref, pc_ref, gt_ref, idx2_ref, bl_ref, ol_ref):
    b = pl.program_id(0)
    px1 = pb_ref[pl.ds(4 * b + 0, 1), :]
    py1 = pb_ref[pl.ds(4 * b + 1, 1), :]
    px2 = pb_ref[pl.ds(4 * b + 2, 1), :]
    py2 = pb_ref[pl.ds(4 * b + 3, 1), :]
    pc = pc_ref[pl.ds(b, 1), :]
    gt = gt_ref[0]
    gx1 = gt[:, 0:1]
    gy1 = gt[:, 1:2]
    gx2 = gt[:, 2:3]
    gy2 = gt[:, 3:4]
    garea = (gx2 - gx1) * (gy2 - gy1)
    best = idx2_ref[0]

    lane = lax.broadcasted_iota(jnp.int32, (_T, _N), 1)
    onehot = lane == best
    colmask = jnp.max(jnp.where(onehot, 1.0, 0.0), axis=0, keepdims=True)
    obj = (jnp.sum(_softplus(pc)) - jnp.sum(colmask * pc)) / float(_N)
    ol_ref[0] = jnp.broadcast_to(obj, (1, 1))

    mx1 = jnp.sum(jnp.where(onehot, px1, 0.0), axis=1, keepdims=True)
    my1 = jnp.sum(jnp.where(onehot, py1, 0.0), axis=1, keepdims=True)
    mx2 = jnp.sum(jnp.where(onehot, px2, 0.0), axis=1, keepdims=True)
    my2 = jnp.sum(jnp.where(onehot, py2, 0.0), axis=1, keepdims=True)

    area1 = (mx2 - mx1) * (my2 - my1)
    left = jnp.maximum(mx1, gx1)
    top = jnp.maximum(my1, gy1)
    right = jnp.minimum(mx2, gx2)
    bottom = jnp.minimum(my2, gy2)
    wh = jnp.maximum(right - left, 0.0) * jnp.maximum(bottom - top, 0.0)
    uni = area1 + garea - wh
    iou_d = wh / (uni + 1e-6)
    cx1 = (mx1 + mx2) * 0.5
    cy1 = (my1 + my2) * 0.5
    cx2 = (gx1 + gx2) * 0.5
    cy2 = (gy1 + gy2) * 0.5
    ex1 = jnp.minimum(mx1, gx1)
    ey1 = jnp.minimum(my1, gy1)
    ex2 = jnp.maximum(mx2, gx2)
    ey2 = jnp.maximum(my2, gy2)
    c_diag = (ex2 - ex1) ** 2 + (ey2 - ey1) ** 2
    center_dist = (cx1 - cx2) ** 2 + (cy1 - cy2) ** 2
    w1 = mx2 - mx1
    h1 = my2 - my1
    w2 = gx2 - gx1
    h2 = gy2 - gy1
    v = 4.0 / (jnp.pi ** 2) * (_atan(w2 / h2) - _atan(w1 / h1)) ** 2
    alpha = v / (1.0 - iou_d + v + 1e-6)
    ciou = iou_d - center_dist / c_diag - alpha * v
    bl_ref[0] = jnp.broadcast_to(jnp.sum(1.0 - ciou) / float(_T), (1, 1))


def _losses(pb, pc, gt, idx2):
    return pl.pallas_call(
        _losses_body,
        grid=(_B,),
        in_specs=[pl.BlockSpec((4 * _B, _N), lambda b: (0, 0)),
                  pl.BlockSpec((_B, _N), lambda b: (0, 0)),
                  pl.BlockSpec((1, _T, 4), lambda b: (b, 0, 0)),
                  pl.BlockSpec((1, _T, 1), lambda b: (b, 0, 0))],
        out_specs=[pl.BlockSpec((1, 1, 1), lambda b: (b, 0, 0)),
                   pl.BlockSpec((1, 1, 1), lambda b: (b, 0, 0))],
        out_shape=[jax.ShapeDtypeStruct((_B, 1, 1), jnp.float32),
                   jax.ShapeDtypeStruct((_B, 1, 1), jnp.float32)],
        compiler_params=pltpu.CompilerParams(
            dimension_semantics=("arbitrary",)),
    )(pb, pc, gt, idx2)


def _gather_cols_sc(table, idx):
    mesh = plsc.VectorSubcoreMesh(core_axis_name="c", subcore_axis_name="s")

    nbuf = 4

    @functools.partial(
        pl.kernel, mesh=mesh,
        out_type=jax.ShapeDtypeStruct((_NW, _C, _BPW), jnp.float32),
        scratch_types=[
            pltpu.VMEM((_BPW,), jnp.int32),
            pltpu.VMEM((nbuf, _C, 128), jnp.float32),
            pltpu.VMEM((_C, _BPW), jnp.float32),
            pltpu.SemaphoreType.DMA,
        ],
        compiler_params=pltpu.CompilerParams(needs_layout_passes=False),
    )
    def k(table_hbm, idx_hbm, out_hbm, idx_v, slab_v, col_v, sem):
        wid = lax.axis_index("s") * 2 + lax.axis_index("c")
        base = wid * _BPW
        pltpu.sync_copy(idx_hbm.at[pl.ds(base, _BPW)], idx_v)
        iv = idx_v[...]
        riota = lax.iota(jnp.int32, 16)

        def fire(j):
            row0 = pl.multiple_of(((base + j) // _T) * _C, _C)
            r = iv[j]
            lt = pl.multiple_of(r - r % 128, 128)
            return pltpu.async_copy(
                table_hbm.at[pl.ds(row0, _C), pl.ds(lt, 128)],
                slab_v.at[j % nbuf], sem)

        def select(j):
            rmv = jnp.full((16,), iv[j] % 128, dtype=jnp.int32)
            for c in range(_C // 16):
                vals = plsc.load_gather(
                    slab_v.at[j % nbuf], [c * 16 + riota, rmv])
                plsc.store_scatter(
                    col_v, [c * 16 + riota,
                            jnp.full((16,), j, dtype=jnp.int32)], vals)

        cps = [fire(j) for j in range(nbuf)]
        for j in range(_BPW):
            cps[j].wait()
            select(j)
            if j + nbuf < _BPW:
                cps.append(fire(j + nbuf))
        pltpu.sync_copy(col_v, out_hbm.at[wid])

    return k(table, idx)


def _finish_body(x_ref, lab_ref, bl_ref, ol_ref, out_ref):
    x = x_ref[...]
    lab = lab_ref[...]
    cls_iota = lax.broadcasted_iota(jnp.int32, x.shape, 1)
    picked = jnp.sum(jnp.where(cls_iota == lab, x, 0.0))
    sp = jnp.sum(_softplus(x))
    cls_total = (sp - picked) / float(_T * _C * _B)
    box_total = jnp.sum(bl_ref[...]) / float(_B)
    obj_total = jnp.sum(ol_ref[...]) / float(_B)
    out_ref[...] = jnp.broadcast_to(
        _LAMBDA_COORD * box_total + obj_total + cls_total, (1, 1))


def _finish(x, labg, bl, ol):
    return pl.pallas_call(
        _finish_body,
        out_shape=jax.ShapeDtypeStruct((1, 1), jnp.float32),
    )(x, labg, bl, ol)


def kernel(pred_boxes, pred_conf, pred_cls, boxes, labels, anchors):
    del anchors
    pb = jnp.swapaxes(pred_boxes, 1, 2).reshape(4 * _B, _N)
    pc = jnp.swapaxes(pred_conf, 1, 2).reshape(_B, _N)
    idx, idx2 = _argmax(pb, boxes)
    table = jnp.swapaxes(pred_cls, 1, 2).reshape(_B * _C, _N)
    cols = _gather_cols_sc(table, idx.reshape(_B * _T))
    bl, ol = _losses(pb, pc, boxes, idx2)
    labg = labels.reshape(_NW, 1, _BPW).astype(jnp.int32)
    out = _finish(cols, labg, bl.reshape(_B, 1), ol.reshape(_B, 1))
    return out.reshape(())


# speedup vs baseline: 2.8235x; 1.0017x over previous
import functools

import jax
import jax.numpy as jnp
from jax import lax
from jax.experimental import pallas as pl
from jax.experimental.pallas import tpu as pltpu
from jax.experimental.pallas import tpu_sc as plsc

_C = 80
_B = 16
_N = 8400
_T = 32
_NW = 32
_BPW = (_B * _T) // _NW
_LAMBDA_COORD = 5.0


def _softplus(x):
    return jnp.maximum(x, 0.0) + jnp.log1p(jnp.exp(-jnp.abs(x)))


_ATAN_COEFFS = (9.999998711639e-01, -3.333252400263e-01, 1.998488468557e-01,
                -1.415480604185e-01, 1.047753919858e-01, -7.194384542314e-02,
                3.934541314624e-02, -1.415234803516e-02, 2.398139012257e-03)


def _atan(x):
    a = jnp.abs(x)
    inv = a > 1.0
    z = jnp.where(inv, 1.0 / a, a)
    t = z * z
    p = jnp.full_like(z, _ATAN_COEFFS[-1])
    for coef in _ATAN_COEFFS[-2::-1]:
        p = p * t + coef
    p = p * z
    r = jnp.where(inv, (jnp.pi / 2.0) - p, p)
    return jnp.sign(x) * r


def _argmax_body(pb_ref, gt_ref, idx_ref, idx2_ref):
    b = pl.program_id(0)
    px1 = pb_ref[pl.ds(4 * b + 0, 1), :]
    py1 = pb_ref[pl.ds(4 * b + 1, 1), :]
    px2 = pb_ref[pl.ds(4 * b + 2, 1), :]
    py2 = pb_ref[pl.ds(4 * b + 3, 1), :]
    gt = gt_ref[0]
    gx1 = gt[:, 0:1]
    gy1 = gt[:, 1:2]
    gx2 = gt[:, 2:3]
    gy2 = gt[:, 3:4]

    parea = (px2 - px1) * (py2 - py1)
    garea = (gx2 - gx1) * (gy2 - gy1)
    ix1 = jnp.maximum(px1, gx1)
    iy1 = jnp.maximum(py1, gy1)
    ix2 = jnp.minimum(px2, gx2)
    iy2 = jnp.minimum(py2, gy2)
    inter = jnp.maximum(ix2 - ix1, 0.0) * jnp.maximum(iy2 - iy1, 0.0)
    union = parea + garea - inter
    iou = inter / (union + 1e-6)

    m = jnp.max(iou, axis=1, keepdims=True)
    lane = lax.broadcasted_iota(jnp.int32, iou.shape, 1)
    best = jnp.min(jnp.where(iou == m, lane, _N - 1), axis=1,
                   keepdims=True)
    idx2_ref[0] = best

    ident = (lax.broadcasted_iota(jnp.int32, (_T, _T), 0)
             == lax.broadcasted_iota(jnp.int32, (_T, _T), 1))
    best_t = jnp.sum(jnp.where(ident, jnp.broadcast_to(best, (_T, _T)), 0),
                     axis=0, keepdims=True)
    idx_ref[0] = best_t


def _argmax(pb, gt):
    return pl.pallas_call(
        _argmax_body,
        grid=(_B,),
        in_specs=[pl.BlockSpec((4 * _B, _N), lambda b: (0, 0)),
                  pl.BlockSpec((1, _T, 4), lambda b: (b, 0, 0))],
        out_specs=[pl.BlockSpec((1, 1, _T), lambda b: (b, 0, 0)),
                   pl.BlockSpec((1, _T, 1), lambda b: (b, 0, 0))],
        out_shape=[jax.ShapeDtypeStruct((_B, 1, _T), jnp.int32),
                   jax.ShapeDtypeStruct((_B, _T, 1), jnp.int32)],
        compiler_params=pltpu.CompilerParams(
            dimension_semantics=("parallel",)),
    )(pb, gt)


def _losses_body(pb_ref, pc_ref, gt_ref, idx2_ref, bl_ref, ol_ref):
    b = pl.program_id(0)
    px1 = pb_ref[pl.ds(4 * b + 0, 1), :]
    py1 = pb_ref[pl.ds(4 * b + 1, 1), :]
    px2 = pb_ref[pl.ds(4 * b + 2, 1), :]
    py2 = pb_ref[pl.ds(4 * b + 3, 1), :]
    pc = pc_ref[pl.ds(b, 1), :]
    gt = gt_ref[0]
    gx1 = gt[:, 0:1]
    gy1 = gt[:, 1:2]
    gx2 = gt[:, 2:3]
    gy2 = gt[:, 3:4]
    garea = (gx2 - gx1) * (gy2 - gy1)
    best = idx2_ref[0]

    lane = lax.broadcasted_iota(jnp.int32, (_T, _N), 1)
    onehot = lane == best
    colmask = jnp.max(jnp.where(onehot, 1.0, 0.0), axis=0, keepdims=True)
    obj = (jnp.sum(_softplus(pc)) - jnp.sum(colmask * pc)) / float(_N)
    ol_ref[0] = jnp.broadcast_to(obj, (1, 1))

    mx1 = jnp.sum(jnp.where(onehot, px1, 0.0), axis=1, keepdims=True)
    my1 = jnp.sum(jnp.where(onehot, py1, 0.0), axis=1, keepdims=True)
    mx2 = jnp.sum(jnp.where(onehot, px2, 0.0), axis=1, keepdims=True)
    my2 = jnp.sum(jnp.where(onehot, py2, 0.0), axis=1, keepdims=True)

    area1 = (mx2 - mx1) * (my2 - my1)
    left = jnp.maximum(mx1, gx1)
    top = jnp.maximum(my1, gy1)
    right = jnp.minimum(mx2, gx2)
    bottom = jnp.minimum(my2, gy2)
    wh = jnp.maximum(right - left, 0.0) * jnp.maximum(bottom - top, 0.0)
    uni = area1 + garea - wh
    iou_d = wh / (uni + 1e-6)
    cx1 = (mx1 + mx2) * 0.5
    cy1 = (my1 + my2) * 0.5
    cx2 = (gx1 + gx2) * 0.5
    cy2 = (gy1 + gy2) * 0.5
    ex1 = jnp.minimum(mx1, gx1)
    ey1 = jnp.minimum(my1, gy1)
    ex2 = jnp.maximum(mx2, gx2)
    ey2 = jnp.maximum(my2, gy2)
    c_diag = (ex2 - ex1) ** 2 + (ey2 - ey1) ** 2
    center_dist = (cx1 - cx2) ** 2 + (cy1 - cy2) ** 2
    w1 = mx2 - mx1
    h1 = my2 - my1
    w2 = gx2 - gx1
    h2 = gy2 - gy1
    v = 4.0 / (jnp.pi ** 2) * (_atan(w2 / h2) - _atan(w1 / h1)) ** 2
    alpha = v / (1.0 - iou_d + v + 1e-6)
    ciou = iou_d - center_dist / c_diag - alpha * v
    bl_ref[0] = jnp.broadcast_to(jnp.sum(1.0 - ciou) / float(_T), (1, 1))


def _losses(pb, pc, gt, idx2):
    return pl.pallas_call(
        _losses_body,
        grid=(_B,),
        in_specs=[pl.BlockSpec((4 * _B, _N), lambda b: (0, 0)),
                  pl.BlockSpec((_B, _N), lambda b: (0, 0)),
                  pl.BlockSpec((1, _T, 4), lambda b: (b, 0, 0)),
                  pl.BlockSpec((1, _T, 1), lambda b: (b, 0, 0))],
        out_specs=[pl.BlockSpec((1, 1, 1), lambda b: (b, 0, 0)),
                   pl.BlockSpec((1, 1, 1), lambda b: (b, 0, 0))],
        out_shape=[jax.ShapeDtypeStruct((_B, 1, 1), jnp.float32),
                   jax.ShapeDtypeStruct((_B, 1, 1), jnp.float32)],
        compiler_params=pltpu.CompilerParams(
            dimension_semantics=("parallel",)),
    )(pb, pc, gt, idx2)


def _gather_cols_sc(table, idx):
    mesh = plsc.VectorSubcoreMesh(core_axis_name="c", subcore_axis_name="s")

    nbuf = 4

    @functools.partial(
        pl.kernel, mesh=mesh,
        out_type=jax.ShapeDtypeStruct((_NW, _C, _BPW), jnp.float32),
        scratch_types=[
            pltpu.VMEM((_BPW,), jnp.int32),
            pltpu.VMEM((nbuf, _C, 128), jnp.float32),
            pltpu.VMEM((_C, _BPW), jnp.float32),
            pltpu.SemaphoreType.DMA,
        ],
        compiler_params=pltpu.CompilerParams(needs_layout_passes=False),
    )
    def k(table_hbm, idx_hbm, out_hbm, idx_v, slab_v, col_v, sem):
        wid = lax.axis_index("s") * 2 + lax.axis_index("c")
        base = wid * _BPW
        pltpu.sync_copy(idx_hbm.at[pl.ds(base, _BPW)], idx_v)
        iv = idx_v[...]
        riota = lax.iota(jnp.int32, 16)

        def fire(j):
            row0 = pl.multiple_of(((base + j) // _T) * _C, _C)
            r = iv[j]
            lt = pl.multiple_of(r - r % 128, 128)
            return pltpu.async_copy(
                table_hbm.at[pl.ds(row0, _C), pl.ds(lt, 128)],
                slab_v.at[j % nbuf], sem)

        def select(j):
            rmv = jnp.full((16,), iv[j] % 128, dtype=jnp.int32)
            for c in range(_C // 16):
                vals = plsc.load_gather(
                    slab_v.at[j % nbuf], [c * 16 + riota, rmv])
                plsc.store_scatter(
                    col_v, [c * 16 + riota,
                            jnp.full((16,), j, dtype=jnp.int32)], vals)

        cps = [fire(j) for j in range(nbuf)]
        for j in range(_BPW):
            cps[j].wait()
            select(j)
            if j + nbuf < _BPW:
                cps.append(fire(j + nbuf))
        pltpu.sync_copy(col_v, out_hbm.at[wid])

    return k(table, idx)


def _finish_body(x_ref, lab_ref, bl_ref, ol_ref, out_ref):
    x = x_ref[...]
    lab = lab_ref[...]
    cls_iota = lax.broadcasted_iota(jnp.int32, x.shape, 1)
    picked = jnp.sum(jnp.where(cls_iota == lab, x, 0.0))
    sp = jnp.sum(_softplus(x))
    cls_total = (sp - picked) / float(_T * _C * _B)
    box_total = jnp.sum(bl_ref[...]) / float(_B)
    obj_total = jnp.sum(ol_ref[...]) / float(_B)
    out_ref[...] = jnp.broadcast_to(
        _LAMBDA_COORD * box_total + obj_total + cls_total, (1, 1))


def _finish(x, labg, bl, ol):
    return pl.pallas_call(
        _finish_body,
        out_shape=jax.ShapeDtypeStruct((1, 1), jnp.float32),
    )(x, labg, bl, ol)


def kernel(pred_boxes, pred_conf, pred_cls, boxes, labels, anchors):
    del anchors
    pb = jnp.swapaxes(pred_boxes, 1, 2).reshape(4 * _B, _N)
    pc = jnp.swapaxes(pred_conf, 1, 2).reshape(_B, _N)
    idx, idx2 = _argmax(pb, boxes)
    table = jnp.swapaxes(pred_cls, 1, 2).reshape(_B * _C, _N)
    cols = _gather_cols_sc(table, idx.reshape(_B * _T))
    bl, ol = _losses(pb, pc, boxes, idx2)
    labg = labels.reshape(_NW, 1, _BPW).astype(jnp.int32)
    out = _finish(cols, labg, bl.reshape(_B, 1), ol.reshape(_B, 1))
    return out.reshape(())


# speedup vs baseline: 3.0755x; 1.0893x over previous
import functools

import jax
import jax.numpy as jnp
from jax import lax
from jax.experimental import pallas as pl
from jax.experimental.pallas import tpu as pltpu
from jax.experimental.pallas import tpu_sc as plsc

_C = 80
_B = 16
_N = 8400
_T = 32
_NW = 32
_BPW = (_B * _T) // _NW
_LAMBDA_COORD = 5.0


def _softplus(x):
    return jnp.maximum(x, 0.0) + jnp.log1p(jnp.exp(-jnp.abs(x)))


_ATAN_COEFFS = (9.999998711639e-01, -3.333252400263e-01, 1.998488468557e-01,
                -1.415480604185e-01, 1.047753919858e-01, -7.194384542314e-02,
                3.934541314624e-02, -1.415234803516e-02, 2.398139012257e-03)


def _atan(x):
    a = jnp.abs(x)
    inv = a > 1.0
    z = jnp.where(inv, 1.0 / a, a)
    t = z * z
    p = jnp.full_like(z, _ATAN_COEFFS[-1])
    for coef in _ATAN_COEFFS[-2::-1]:
        p = p * t + coef
    p = p * z
    r = jnp.where(inv, (jnp.pi / 2.0) - p, p)
    return jnp.sign(x) * r


def _argmax_body(pb_ref, gt_ref, idx_ref, idx2_ref):
    px1 = pb_ref[0, 0:1, :]
    py1 = pb_ref[0, 1:2, :]
    px2 = pb_ref[0, 2:3, :]
    py2 = pb_ref[0, 3:4, :]
    gt = gt_ref[0]
    gx1 = gt[:, 0:1]
    gy1 = gt[:, 1:2]
    gx2 = gt[:, 2:3]
    gy2 = gt[:, 3:4]

    parea = (px2 - px1) * (py2 - py1)
    garea = (gx2 - gx1) * (gy2 - gy1)
    ix1 = jnp.maximum(px1, gx1)
    iy1 = jnp.maximum(py1, gy1)
    ix2 = jnp.minimum(px2, gx2)
    iy2 = jnp.minimum(py2, gy2)
    inter = jnp.maximum(ix2 - ix1, 0.0) * jnp.maximum(iy2 - iy1, 0.0)
    union = parea + garea - inter
    iou = inter / (union + 1e-6)

    m = jnp.max(iou, axis=1, keepdims=True)
    lane = lax.broadcasted_iota(jnp.int32, iou.shape, 1)
    best = jnp.min(jnp.where(iou == m, lane, _N - 1), axis=1,
                   keepdims=True)
    idx2_ref[0] = best

    ident = (lax.broadcasted_iota(jnp.int32, (_T, _T), 0)
             == lax.broadcasted_iota(jnp.int32, (_T, _T), 1))
    best_t = jnp.sum(jnp.where(ident, jnp.broadcast_to(best, (_T, _T)), 0),
                     axis=0, keepdims=True)
    idx_ref[0] = best_t


def _argmax(pb, gt):
    return pl.pallas_call(
        _argmax_body,
        grid=(_B,),
        in_specs=[pl.BlockSpec((1, 4, _N), lambda b: (b, 0, 0)),
                  pl.BlockSpec((1, _T, 4), lambda b: (b, 0, 0))],
        out_specs=[pl.BlockSpec((1, 1, _T), lambda b: (b, 0, 0)),
                   pl.BlockSpec((1, _T, 1), lambda b: (b, 0, 0))],
        out_shape=[jax.ShapeDtypeStruct((_B, 1, _T), jnp.int32),
                   jax.ShapeDtypeStruct((_B, _T, 1), jnp.int32)],
        compiler_params=pltpu.CompilerParams(
            dimension_semantics=("parallel",)),
    )(pb, gt)


def _losses_body(pb_ref, pc_ref, gt_ref, idx2_ref, bl_ref, ol_ref):
    px1 = pb_ref[0, 0:1, :]
    py1 = pb_ref[0, 1:2, :]
    px2 = pb_ref[0, 2:3, :]
    py2 = pb_ref[0, 3:4, :]
    pc = pc_ref[0]
    gt = gt_ref[0]
    gx1 = gt[:, 0:1]
    gy1 = gt[:, 1:2]
    gx2 = gt[:, 2:3]
    gy2 = gt[:, 3:4]
    garea = (gx2 - gx1) * (gy2 - gy1)
    best = idx2_ref[0]

    lane = lax.broadcasted_iota(jnp.int32, (_T, _N), 1)
    onehot = lane == best
    colmask = jnp.max(jnp.where(onehot, 1.0, 0.0), axis=0, keepdims=True)
    obj = (jnp.sum(_softplus(pc)) - jnp.sum(colmask * pc)) / float(_N)
    ol_ref[0] = jnp.broadcast_to(obj, (1, 1))

    mx1 = jnp.sum(jnp.where(onehot, px1, 0.0), axis=1, keepdims=True)
    my1 = jnp.sum(jnp.where(onehot, py1, 0.0), axis=1, keepdims=True)
    mx2 = jnp.sum(jnp.where(onehot, px2, 0.0), axis=1, keepdims=True)
    my2 = jnp.sum(jnp.where(onehot, py2, 0.0), axis=1, keepdims=True)

    area1 = (mx2 - mx1) * (my2 - my1)
    left = jnp.maximum(mx1, gx1)
    top = jnp.maximum(my1, gy1)
    right = jnp.minimum(mx2, gx2)
    bottom = jnp.minimum(my2, gy2)
    wh = jnp.maximum(right - left, 0.0) * jnp.maximum(bottom - top, 0.0)
    uni = area1 + garea - wh
    iou_d = wh / (uni + 1e-6)
    cx1 = (mx1 + mx2) * 0.5
    cy1 = (my1 + my2) * 0.5
    cx2 = (gx1 + gx2) * 0.5
    cy2 = (gy1 + gy2) * 0.5
    ex1 = jnp.minimum(mx1, gx1)
    ey1 = jnp.minimum(my1, gy1)
    ex2 = jnp.maximum(mx2, gx2)
    ey2 = jnp.maximum(my2, gy2)
    c_diag = (ex2 - ex1) ** 2 + (ey2 - ey1) ** 2
    center_dist = (cx1 - cx2) ** 2 + (cy1 - cy2) ** 2
    w1 = mx2 - mx1
    h1 = my2 - my1
    w2 = gx2 - gx1
    h2 = gy2 - gy1
    v = 4.0 / (jnp.pi ** 2) * (_atan(w2 / h2) - _atan(w1 / h1)) ** 2
    alpha = v / (1.0 - iou_d + v + 1e-6)
    ciou = iou_d - center_dist / c_diag - alpha * v
    bl_ref[0] = jnp.broadcast_to(jnp.sum(1.0 - ciou) / float(_T), (1, 1))


def _losses(pb, pc, gt, idx2):
    return pl.pallas_call(
        _losses_body,
        grid=(_B,),
        in_specs=[pl.BlockSpec((1, 4, _N), lambda b: (b, 0, 0)),
                  pl.BlockSpec((1, 1, _N), lambda b: (b, 0, 0)),
                  pl.BlockSpec((1, _T, 4), lambda b: (b, 0, 0)),
                  pl.BlockSpec((1, _T, 1), lambda b: (b, 0, 0))],
        out_specs=[pl.BlockSpec((1, 1, 1), lambda b: (b, 0, 0)),
                   pl.BlockSpec((1, 1, 1), lambda b: (b, 0, 0))],
        out_shape=[jax.ShapeDtypeStruct((_B, 1, 1), jnp.float32),
                   jax.ShapeDtypeStruct((_B, 1, 1), jnp.float32)],
        compiler_params=pltpu.CompilerParams(
            dimension_semantics=("parallel",)),
    )(pb, pc, gt, idx2)


def _gather_cols_sc(table, idx):
    mesh = plsc.VectorSubcoreMesh(core_axis_name="c", subcore_axis_name="s")

    nbuf = 4

    @functools.partial(
        pl.kernel, mesh=mesh,
        out_type=jax.ShapeDtypeStruct((_NW, _C, _BPW), jnp.float32),
        scratch_types=[
            pltpu.VMEM((_BPW,), jnp.int32),
            pltpu.VMEM((nbuf, _C, 128), jnp.float32),
            pltpu.VMEM((_C, _BPW), jnp.float32),
            pltpu.SemaphoreType.DMA,
        ],
        compiler_params=pltpu.CompilerParams(needs_layout_passes=False),
    )
    def k(table_hbm, idx_hbm, out_hbm, idx_v, slab_v, col_v, sem):
        wid = lax.axis_index("s") * 2 + lax.axis_index("c")
        base = wid * _BPW
        pltpu.sync_copy(idx_hbm.at[pl.ds(base, _BPW)], idx_v)
        iv = idx_v[...]
        riota = lax.iota(jnp.int32, 16)

        def fire(j):
            row0 = pl.multiple_of(((base + j) // _T) * _C, _C)
            r = iv[j]
            lt = pl.multiple_of(r - r % 128, 128)
            return pltpu.async_copy(
                table_hbm.at[pl.ds(row0, _C), pl.ds(lt, 128)],
                slab_v.at[j % nbuf], sem)

        def select(j):
            rmv = jnp.full((16,), iv[j] % 128, dtype=jnp.int32)
            for c in range(_C // 16):
                vals = plsc.load_gather(
                    slab_v.at[j % nbuf], [c * 16 + riota, rmv])
                plsc.store_scatter(
                    col_v, [c * 16 + riota,
                            jnp.full((16,), j, dtype=jnp.int32)], vals)

        cps = [fire(j) for j in range(nbuf)]
        for j in range(_BPW):
            cps[j].wait()
            select(j)
            if j + nbuf < _BPW:
                cps.append(fire(j + nbuf))
        pltpu.sync_copy(col_v, out_hbm.at[wid])

    return k(table, idx)


def _finish_body(x_ref, lab_ref, bl_ref, ol_ref, out_ref):
    x = x_ref[...]
    lab = lab_ref[...]
    cls_iota = lax.broadcasted_iota(jnp.int32, x.shape, 1)
    picked = jnp.sum(jnp.where(cls_iota == lab, x, 0.0))
    sp = jnp.sum(_softplus(x))
    cls_total = (sp - picked) / float(_T * _C * _B)
    box_total = jnp.sum(bl_ref[...]) / float(_B)
    obj_total = jnp.sum(ol_ref[...]) / float(_B)
    out_ref[...] = jnp.broadcast_to(
        _LAMBDA_COORD * box_total + obj_total + cls_total, (1, 1))


def _finish(x, labg, bl, ol):
    return pl.pallas_call(
        _finish_body,
        out_shape=jax.ShapeDtypeStruct((1, 1), jnp.float32),
    )(x, labg, bl, ol)


def kernel(pred_boxes, pred_conf, pred_cls, boxes, labels, anchors):
    del anchors
    pb = jnp.swapaxes(pred_boxes, 1, 2)
    pc = jnp.swapaxes(pred_conf, 1, 2)
    idx, idx2 = _argmax(pb, boxes)
    table = jnp.swapaxes(pred_cls, 1, 2).reshape(_B * _C, _N)
    cols = _gather_cols_sc(table, idx.reshape(_B * _T))
    bl, ol = _losses(pb, pc, boxes, idx2)
    labg = labels.reshape(_NW, 1, _BPW).astype(jnp.int32)
    out = _finish(cols, labg, bl.reshape(_B, 1), ol.reshape(_B, 1))
    return out.reshape(())


# speedup vs baseline: 3.4276x; 1.1145x over previous
import functools

import jax
import jax.numpy as jnp
from jax import lax
from jax.experimental import pallas as pl
from jax.experimental.pallas import tpu as pltpu
from jax.experimental.pallas import tpu_sc as plsc

_C = 80
_B = 16
_N = 8400
_T = 32
_NW = 32
_BPW = (_B * _T) // _NW
_LAMBDA_COORD = 5.0


def _softplus(x):
    return jnp.maximum(x, 0.0) + jnp.log1p(jnp.exp(-jnp.abs(x)))


_ATAN_COEFFS = (9.999998711639e-01, -3.333252400263e-01, 1.998488468557e-01,
                -1.415480604185e-01, 1.047753919858e-01, -7.194384542314e-02,
                3.934541314624e-02, -1.415234803516e-02, 2.398139012257e-03)


def _atan(x):
    a = jnp.abs(x)
    inv = a > 1.0
    z = jnp.where(inv, 1.0 / a, a)
    t = z * z
    p = jnp.full_like(z, _ATAN_COEFFS[-1])
    for coef in _ATAN_COEFFS[-2::-1]:
        p = p * t + coef
    p = p * z
    r = jnp.where(inv, (jnp.pi / 2.0) - p, p)
    return jnp.sign(x) * r


def _argmax_body(pb_ref, gt_ref, idx_ref, idx2_ref):
  for i in range(2):
    px1 = pb_ref[i, 0:1, :]
    py1 = pb_ref[i, 1:2, :]
    px2 = pb_ref[i, 2:3, :]
    py2 = pb_ref[i, 3:4, :]
    gt = gt_ref[i]
    gx1 = gt[:, 0:1]
    gy1 = gt[:, 1:2]
    gx2 = gt[:, 2:3]
    gy2 = gt[:, 3:4]

    parea = (px2 - px1) * (py2 - py1)
    garea = (gx2 - gx1) * (gy2 - gy1)
    ix1 = jnp.maximum(px1, gx1)
    iy1 = jnp.maximum(py1, gy1)
    ix2 = jnp.minimum(px2, gx2)
    iy2 = jnp.minimum(py2, gy2)
    inter = jnp.maximum(ix2 - ix1, 0.0) * jnp.maximum(iy2 - iy1, 0.0)
    union = parea + garea - inter
    iou = inter / (union + 1e-6)

    m = jnp.max(iou, axis=1, keepdims=True)
    lane = lax.broadcasted_iota(jnp.int32, iou.shape, 1)
    best = jnp.min(jnp.where(iou == m, lane, _N - 1), axis=1,
                   keepdims=True)
    idx2_ref[i] = best

    ident = (lax.broadcasted_iota(jnp.int32, (_T, _T), 0)
             == lax.broadcasted_iota(jnp.int32, (_T, _T), 1))
    best_t = jnp.sum(jnp.where(ident, jnp.broadcast_to(best, (_T, _T)), 0),
                     axis=0, keepdims=True)
    idx_ref[i] = best_t


def _argmax(pb, gt):
    return pl.pallas_call(
        _argmax_body,
        grid=(_B // 2,),
        in_specs=[pl.BlockSpec((2, 4, _N), lambda b: (b, 0, 0)),
                  pl.BlockSpec((2, _T, 4), lambda b: (b, 0, 0))],
        out_specs=[pl.BlockSpec((2, 1, _T), lambda b: (b, 0, 0)),
                   pl.BlockSpec((2, _T, 1), lambda b: (b, 0, 0))],
        out_shape=[jax.ShapeDtypeStruct((_B, 1, _T), jnp.int32),
                   jax.ShapeDtypeStruct((_B, _T, 1), jnp.int32)],
        compiler_params=pltpu.CompilerParams(
            dimension_semantics=("parallel",)),
    )(pb, gt)


def _losses_body(pb_ref, pc_ref, gt_ref, idx2_ref, bl_ref, ol_ref):
  for i in range(2):
    px1 = pb_ref[i, 0:1, :]
    py1 = pb_ref[i, 1:2, :]
    px2 = pb_ref[i, 2:3, :]
    py2 = pb_ref[i, 3:4, :]
    pc = pc_ref[i]
    gt = gt_ref[i]
    gx1 = gt[:, 0:1]
    gy1 = gt[:, 1:2]
    gx2 = gt[:, 2:3]
    gy2 = gt[:, 3:4]
    garea = (gx2 - gx1) * (gy2 - gy1)
    best = idx2_ref[i]

    lane = lax.broadcasted_iota(jnp.int32, (_T, _N), 1)
    onehot = lane == best
    colmask = jnp.max(jnp.where(onehot, 1.0, 0.0), axis=0, keepdims=True)
    obj = (jnp.sum(_softplus(pc)) - jnp.sum(colmask * pc)) / float(_N)
    ol_ref[i] = jnp.broadcast_to(obj, (1, 1))

    mx1 = jnp.sum(jnp.where(onehot, px1, 0.0), axis=1, keepdims=True)
    my1 = jnp.sum(jnp.where(onehot, py1, 0.0), axis=1, keepdims=True)
    mx2 = jnp.sum(jnp.where(onehot, px2, 0.0), axis=1, keepdims=True)
    my2 = jnp.sum(jnp.where(onehot, py2, 0.0), axis=1, keepdims=True)

    area1 = (mx2 - mx1) * (my2 - my1)
    left = jnp.maximum(mx1, gx1)
    top = jnp.maximum(my1, gy1)
    right = jnp.minimum(mx2, gx2)
    bottom = jnp.minimum(my2, gy2)
    wh = jnp.maximum(right - left, 0.0) * jnp.maximum(bottom - top, 0.0)
    uni = area1 + garea - wh
    iou_d = wh / (uni + 1e-6)
    cx1 = (mx1 + mx2) * 0.5
    cy1 = (my1 + my2) * 0.5
    cx2 = (gx1 + gx2) * 0.5
    cy2 = (gy1 + gy2) * 0.5
    ex1 = jnp.minimum(mx1, gx1)
    ey1 = jnp.minimum(my1, gy1)
    ex2 = jnp.maximum(mx2, gx2)
    ey2 = jnp.maximum(my2, gy2)
    c_diag = (ex2 - ex1) ** 2 + (ey2 - ey1) ** 2
    center_dist = (cx1 - cx2) ** 2 + (cy1 - cy2) ** 2
    w1 = mx2 - mx1
    h1 = my2 - my1
    w2 = gx2 - gx1
    h2 = gy2 - gy1
    v = 4.0 / (jnp.pi ** 2) * (_atan(w2 / h2) - _atan(w1 / h1)) ** 2
    alpha = v / (1.0 - iou_d + v + 1e-6)
    ciou = iou_d - center_dist / c_diag - alpha * v
    bl_ref[i] = jnp.broadcast_to(jnp.sum(1.0 - ciou) / float(_T), (1, 1))


def _losses(pb, pc, gt, idx2):
    return pl.pallas_call(
        _losses_body,
        grid=(_B // 2,),
        in_specs=[pl.BlockSpec((2, 4, _N), lambda b: (b, 0, 0)),
                  pl.BlockSpec((2, 1, _N), lambda b: (b, 0, 0)),
                  pl.BlockSpec((2, _T, 4), lambda b: (b, 0, 0)),
                  pl.BlockSpec((2, _T, 1), lambda b: (b, 0, 0))],
        out_specs=[pl.BlockSpec((2, 1, 1), lambda b: (b, 0, 0)),
                   pl.BlockSpec((2, 1, 1), lambda b: (b, 0, 0))],
        out_shape=[jax.ShapeDtypeStruct((_B, 1, 1), jnp.float32),
                   jax.ShapeDtypeStruct((_B, 1, 1), jnp.float32)],
        compiler_params=pltpu.CompilerParams(
            dimension_semantics=("parallel",)),
    )(pb, pc, gt, idx2)


def _gather_cols_sc(table, idx):
    mesh = plsc.VectorSubcoreMesh(core_axis_name="c", subcore_axis_name="s")

    nbuf = 4

    @functools.partial(
        pl.kernel, mesh=mesh,
        out_type=jax.ShapeDtypeStruct((_NW, _C, _BPW), jnp.float32),
        scratch_types=[
            pltpu.VMEM((_BPW,), jnp.int32),
            pltpu.VMEM((nbuf, _C, 128), jnp.float32),
            pltpu.VMEM((_C, _BPW), jnp.float32),
            pltpu.SemaphoreType.DMA,
        ],
        compiler_params=pltpu.CompilerParams(needs_layout_passes=False),
    )
    def k(table_hbm, idx_hbm, out_hbm, idx_v, slab_v, col_v, sem):
        wid = lax.axis_index("s") * 2 + lax.axis_index("c")
        base = wid * _BPW
        pltpu.sync_copy(idx_hbm.at[pl.ds(base, _BPW)], idx_v)
        iv = idx_v[...]
        riota = lax.iota(jnp.int32, 16)

        def fire(j):
            row0 = pl.multiple_of(((base + j) // _T) * _C, _C)
            r = iv[j]
            lt = pl.multiple_of(r - r % 128, 128)
            return pltpu.async_copy(
                table_hbm.at[pl.ds(row0, _C), pl.ds(lt, 128)],
                slab_v.at[j % nbuf], sem)

        def select(j):
            rmv = jnp.full((16,), iv[j] % 128, dtype=jnp.int32)
            for c in range(_C // 16):
                vals = plsc.load_gather(
                    slab_v.at[j % nbuf], [c * 16 + riota, rmv])
                plsc.store_scatter(
                    col_v, [c * 16 + riota,
                            jnp.full((16,), j, dtype=jnp.int32)], vals)

        cps = [fire(j) for j in range(nbuf)]
        for j in range(_BPW):
            cps[j].wait()
            select(j)
            if j + nbuf < _BPW:
                cps.append(fire(j + nbuf))
        pltpu.sync_copy(col_v, out_hbm.at[wid])

    return k(table, idx)


def _finish_body(x_ref, lab_ref, bl_ref, ol_ref, out_ref):
    x = x_ref[...]
    lab = lab_ref[...]
    cls_iota = lax.broadcasted_iota(jnp.int32, x.shape, 1)
    picked = jnp.sum(jnp.where(cls_iota == lab, x, 0.0))
    sp = jnp.sum(_softplus(x))
    cls_total = (sp - picked) / float(_T * _C * _B)
    box_total = jnp.sum(bl_ref[...]) / float(_B)
    obj_total = jnp.sum(ol_ref[...]) / float(_B)
    out_ref[...] = jnp.broadcast_to(
        _LAMBDA_COORD * box_total + obj_total + cls_total, (1, 1))


def _finish(x, labg, bl, ol):
    return pl.pallas_call(
        _finish_body,
        out_shape=jax.ShapeDtypeStruct((1, 1), jnp.float32),
    )(x, labg, bl, ol)


def kernel(pred_boxes, pred_conf, pred_cls, boxes, labels, anchors):
    del anchors
    pb = jnp.swapaxes(pred_boxes, 1, 2)
    pc = jnp.swapaxes(pred_conf, 1, 2)
    idx, idx2 = _argmax(pb, boxes)
    table = jnp.swapaxes(pred_cls, 1, 2).reshape(_B * _C, _N)
    cols = _gather_cols_sc(table, idx.reshape(_B * _T))
    bl, ol = _losses(pb, pc, boxes, idx2)
    labg = labels.reshape(_NW, 1, _BPW).astype(jnp.int32)
    out = _finish(cols, labg, bl.reshape(_B, 1), ol.reshape(_B, 1))
    return out.reshape(())
